# baseline (speedup 1.0000x reference)
import jax
import jax.numpy as jnp
from jax import lax
from jax.experimental import pallas as pl
from jax.experimental.pallas import tpu as pltpu
from jax.experimental.pallas import tpu_sc as plsc

B, L, N, S = 8, 2048, 1000, 4
P = B * L
NCORES = 2
NSUB = 16
NW = NCORES * NSUB
PW = P // NW
LANES = 16
CN = 128
OFFS = [0, 128, 256, 384, 512, 640, 768, 872]
WINS = [(0, 128), (128, 256), (256, 384), (384, 512),
        (512, 640), (640, 768), (768, 896), (896, 1000)]
IPC = PW // LANES
W_PER_USER = L // PW


def _log1p_poly(z):
    t = z / (z + 2.0)
    t2 = t * t
    p = 1.0 / 7.0 + t2 * (1.0 / 9.0)
    p = 1.0 / 5.0 + t2 * p
    p = 1.0 / 3.0 + t2 * p
    return 2.0 * t * (1.0 + t2 * p)


def _log_sigmoid(d):
    z = jnp.exp(-jnp.abs(d))
    return jnp.minimum(d, 0.0) - _log1p_poly(z)


def _body(scores, labels, negs, xlens, parts,
          lab_v, negid_v, nid_sv, xl_v, chunk_v, pv_v, nv_v, res_v, sem):
    wid = lax.axis_index("s") * NCORES + lax.axis_index("c")
    base = wid * PW
    b_user = lax.div(wid, W_PER_USER)
    l0 = lax.rem(wid, W_PER_USER) * PW
    iota = lax.iota(jnp.int32, LANES)

    pltpu.sync_copy(labels.at[pl.ds(base, PW)], lab_v)
    pltpu.sync_copy(negs.at[pl.ds(base * S, PW * S)], negid_v)
    pltpu.sync_copy(xlens, xl_v)

    def stage(i, carry):
        for s in range(S):
            nid = plsc.load_gather(negid_v, [(i * LANES + iota) * S + s])
            nid_sv[s, pl.ds(i * LANES, LANES)] = nid
        return carry

    lax.fori_loop(0, IPC, stage, 0)

    xlen = xl_v[wid]
    CB = 128
    for k in range(PW // CB):
        alive = jnp.logical_and(iota >= 0, (l0 + k * CB) < xlen)
        n_alive = plsc.all_reduce_population_count(alive)

        @pl.when(n_alive[0] > 0)
        def _(k=k):
            for g in range(len(OFFS)):
                rb = OFFS[g]
                w0, w1 = WINS[g]
                pltpu.sync_copy(
                    scores.at[pl.ds(b_user * N + rb, CN),
                              pl.ds(l0 + k * CB, CB)], chunk_v)

                def harvest(i, carry):
                    lcol = i * LANES + iota - k * CB
                    lab16 = lab_v[pl.ds(i * LANES, LANES)]
                    m = jnp.logical_and(lab16 >= w0, lab16 < w1)
                    nl = jnp.where(m, lab16 - rb, 0)
                    v = plsc.load_gather(chunk_v, [nl, lcol])
                    pv_v[pl.ds(i * LANES, LANES)] = jnp.where(
                        m, v, pv_v[pl.ds(i * LANES, LANES)])
                    for s in range(S):
                        nid16 = nid_sv[s, pl.ds(i * LANES, LANES)]
                        ms = jnp.logical_and(nid16 >= w0, nid16 < w1)
                        nls = jnp.where(ms, nid16 - rb, 0)
                        vs = plsc.load_gather(chunk_v, [nls, lcol])
                        nv_v[s, pl.ds(i * LANES, LANES)] = jnp.where(
                            ms, vs, nv_v[s, pl.ds(i * LANES, LANES)])
                    return carry

                lax.fori_loop(k * (CB // LANES), (k + 1) * (CB // LANES),
                              harvest, 0)

    xlenf = jnp.maximum(xlen, 1).astype(jnp.float32)
    scale = (-1.0 / B) / xlenf

    def comp(i, acc):
        pos = pv_v[pl.ds(i * LANES, LANES)]
        ls = jnp.zeros((LANES,), jnp.float32)
        for s in range(S):
            neg = nv_v[s, pl.ds(i * LANES, LANES)]
            ls = ls + _log_sigmoid(pos - neg)
        l_pos = l0 + i * LANES + iota
        return acc + jnp.where(l_pos < xlen, ls, 0.0)

    acc = lax.fori_loop(0, IPC, comp, jnp.zeros((LANES,), jnp.float32))
    res_v[...] = acc * scale
    pltpu.sync_copy(res_v, parts.at[wid])


def _body2(parts, out, all_v, out_v):
    wid = lax.axis_index("s")
    iota = lax.iota(jnp.int32, LANES)

    @pl.when(wid == 0)
    def _():
        pltpu.sync_copy(parts, all_v)
        tot = jnp.zeros((LANES,), jnp.float32)
        for w in range(NW):
            tot = tot + all_v[w]
        total = tot[0]
        for l in range(1, LANES):
            total = total + tot[l]
        out_v[...] = jnp.where(iota == 0, total, 0.0)
        pltpu.sync_copy(out_v.at[pl.ds(0, 8)], out)


@jax.jit
def _bpr_loss(output, labels, x_lens, neg_ids):
    scores = jnp.transpose(output, (0, 2, 1)).reshape(B * N, L)
    lab = labels.reshape(-1)
    neg = neg_ids.reshape(-1)
    xl_rep = jnp.broadcast_to(
        jnp.repeat(x_lens.astype(jnp.int32), W_PER_USER)[:, None],
        (NW, LANES))
    mesh = plsc.VectorSubcoreMesh(
        core_axis_name="c", subcore_axis_name="s", num_cores=NCORES)
    params = pltpu.CompilerParams(needs_layout_passes=False)
    parts = pl.kernel(
        _body,
        out_type=jax.ShapeDtypeStruct((NW, LANES), jnp.float32),
        mesh=mesh,
        scratch_types=[
            pltpu.VMEM((PW,), jnp.int32),
            pltpu.VMEM((PW * S,), jnp.int32),
            pltpu.VMEM((S, PW), jnp.int32),
            pltpu.VMEM((NW, LANES), jnp.int32),
            pltpu.VMEM((CN, 128), jnp.float32),
            pltpu.VMEM((PW,), jnp.float32),
            pltpu.VMEM((S, PW), jnp.float32),
            pltpu.VMEM((LANES,), jnp.float32),
            pltpu.SemaphoreType.DMA,
        ],
        compiler_params=params,
    )(scores, lab, neg, xl_rep)
    mesh2 = plsc.VectorSubcoreMesh(
        core_axis_name="c", subcore_axis_name="s", num_cores=1)
    res = pl.kernel(
        _body2,
        out_type=jax.ShapeDtypeStruct((8,), jnp.float32),
        mesh=mesh2,
        scratch_types=[
            pltpu.VMEM((NW, LANES), jnp.float32),
            pltpu.VMEM((LANES,), jnp.float32),
        ],
        compiler_params=params,
    )(parts)
    return res[0:1]


def kernel(output, labels, x_lens, uids, neg_ids):
    return _bpr_loss(output, labels, x_lens, neg_ids)


# speedup vs baseline: 1.5227x; 1.5227x over previous
import jax
import jax.numpy as jnp
from jax import lax
from jax.experimental import pallas as pl
from jax.experimental.pallas import tpu as pltpu
from jax.experimental.pallas import tpu_sc as plsc

B, L, N, S = 8, 2048, 1000, 4
P = B * L
NCORES = 2
NSUB = 16
NW = NCORES * NSUB
PW = P // NW
LANES = 16
CN = 112
OFFS = [0, 112, 224, 336, 448, 560, 672, 784, 888]
WINS = [(0, 112), (112, 224), (224, 336), (336, 448), (448, 560),
        (560, 672), (672, 784), (784, 896), (896, 1000)]
IPC = PW // LANES
W_PER_USER = L // PW


def _log1p_poly(z):
    t = z / (z + 2.0)
    t2 = t * t
    p = 1.0 / 7.0 + t2 * (1.0 / 9.0)
    p = 1.0 / 5.0 + t2 * p
    p = 1.0 / 3.0 + t2 * p
    return 2.0 * t * (1.0 + t2 * p)


def _log_sigmoid(d):
    z = jnp.exp(-jnp.abs(d))
    return jnp.minimum(d, 0.0) - _log1p_poly(z)


def _body(scores, labels, negs, xlens, parts,
          lab_v, negid_v, nid_sv, xl_v, chunk_v, chunk2_v, pv_v, nv_v,
          res_v, sem):
    wid = lax.axis_index("s") * NCORES + lax.axis_index("c")
    base = wid * PW
    b_user = lax.div(wid, W_PER_USER)
    l0 = lax.rem(wid, W_PER_USER) * PW
    iota = lax.iota(jnp.int32, LANES)

    pltpu.sync_copy(labels.at[pl.ds(base, PW)], lab_v)
    pltpu.sync_copy(negs.at[pl.ds(base * S, PW * S)], negid_v)
    pltpu.sync_copy(xlens, xl_v)

    def stage(i, carry):
        for s in range(S):
            nid = plsc.load_gather(negid_v, [(i * LANES + iota) * S + s])
            nid_sv[s, pl.ds(i * LANES, LANES)] = nid
        return carry

    lax.fori_loop(0, IPC, stage, 0)

    xlen = xl_v[wid]
    alive = jnp.logical_and(iota >= 0, l0 < xlen)
    n_alive = plsc.all_reduce_population_count(alive)

    @pl.when(n_alive[0] > 0)
    def _():
        bufs = [chunk_v, chunk2_v]
        NWIN = len(OFFS)

        def start(g):
            return pltpu.async_copy(
                scores.at[pl.ds(b_user * N + OFFS[g], CN), pl.ds(l0, PW)],
                bufs[g % 2], sem)

        start(0)
        for g in range(NWIN):
            if g + 1 < NWIN:
                start(g + 1)
            pltpu.make_async_copy(
                scores.at[pl.ds(b_user * N + OFFS[g], CN), pl.ds(l0, PW)],
                bufs[g % 2], sem).wait()
            rb = OFFS[g]
            w0, w1 = WINS[g]
            buf = bufs[g % 2]

            def harvest(i, carry, buf=buf, rb=rb, w0=w0, w1=w1):
                lcol = i * LANES + iota
                lab16 = lab_v[pl.ds(i * LANES, LANES)]
                m = jnp.logical_and(lab16 >= w0, lab16 < w1)
                nl = jnp.where(m, lab16 - rb, 0)
                v = plsc.load_gather(buf, [nl, lcol])
                pv_v[pl.ds(i * LANES, LANES)] = jnp.where(
                    m, v, pv_v[pl.ds(i * LANES, LANES)])
                for s in range(S):
                    nid16 = nid_sv[s, pl.ds(i * LANES, LANES)]
                    ms = jnp.logical_and(nid16 >= w0, nid16 < w1)
                    nls = jnp.where(ms, nid16 - rb, 0)
                    vs = plsc.load_gather(buf, [nls, lcol])
                    nv_v[s, pl.ds(i * LANES, LANES)] = jnp.where(
                        ms, vs, nv_v[s, pl.ds(i * LANES, LANES)])
                return carry

            lax.fori_loop(0, IPC, harvest, 0)

    xlenf = jnp.maximum(xlen, 1).astype(jnp.float32)
    scale = (-1.0 / B) / xlenf

    def comp(i, acc):
        pos = pv_v[pl.ds(i * LANES, LANES)]
        ls = jnp.zeros((LANES,), jnp.float32)
        for s in range(S):
            neg = nv_v[s, pl.ds(i * LANES, LANES)]
            ls = ls + _log_sigmoid(pos - neg)
        l_pos = l0 + i * LANES + iota
        return acc + jnp.where(l_pos < xlen, ls, 0.0)

    acc = lax.fori_loop(0, IPC, comp, jnp.zeros((LANES,), jnp.float32))
    res_v[...] = acc * scale
    pltpu.sync_copy(res_v, parts.at[wid])


def _body2(parts, out, all_v, out_v):
    wid = lax.axis_index("s")
    iota = lax.iota(jnp.int32, LANES)

    @pl.when(wid == 0)
    def _():
        pltpu.sync_copy(parts, all_v)
        tot = jnp.zeros((LANES,), jnp.float32)
        for w in range(NW):
            tot = tot + all_v[w]
        total = tot[0]
        for l in range(1, LANES):
            total = total + tot[l]
        out_v[...] = jnp.where(iota == 0, total, 0.0)
        pltpu.sync_copy(out_v.at[pl.ds(0, 8)], out)


@jax.jit
def _bpr_loss(output, labels, x_lens, neg_ids):
    scores = jnp.transpose(output, (0, 2, 1)).reshape(B * N, L)
    lab = labels.reshape(-1)
    neg = neg_ids.reshape(-1)
    xl_rep = jnp.broadcast_to(
        jnp.repeat(x_lens.astype(jnp.int32), W_PER_USER)[:, None],
        (NW, LANES))
    mesh = plsc.VectorSubcoreMesh(
        core_axis_name="c", subcore_axis_name="s", num_cores=NCORES)
    params = pltpu.CompilerParams(needs_layout_passes=False)
    parts = pl.kernel(
        _body,
        out_type=jax.ShapeDtypeStruct((NW, LANES), jnp.float32),
        mesh=mesh,
        scratch_types=[
            pltpu.VMEM((PW,), jnp.int32),
            pltpu.VMEM((PW * S,), jnp.int32),
            pltpu.VMEM((S, PW), jnp.int32),
            pltpu.VMEM((NW, LANES), jnp.int32),
            pltpu.VMEM((CN, PW), jnp.float32),
            pltpu.VMEM((CN, PW), jnp.float32),
            pltpu.VMEM((PW,), jnp.float32),
            pltpu.VMEM((S, PW), jnp.float32),
            pltpu.VMEM((LANES,), jnp.float32),
            pltpu.SemaphoreType.DMA,
        ],
        compiler_params=params,
    )(scores, lab, neg, xl_rep)
    mesh2 = plsc.VectorSubcoreMesh(
        core_axis_name="c", subcore_axis_name="s", num_cores=1)
    res = pl.kernel(
        _body2,
        out_type=jax.ShapeDtypeStruct((8,), jnp.float32),
        mesh=mesh2,
        scratch_types=[
            pltpu.VMEM((NW, LANES), jnp.float32),
            pltpu.VMEM((LANES,), jnp.float32),
        ],
        compiler_params=params,
    )(parts)
    return res[0:1]


def kernel(output, labels, x_lens, uids, neg_ids):
    return _bpr_loss(output, labels, x_lens, neg_ids)


# speedup vs baseline: 2.0454x; 1.3433x over previous
import jax
import jax.numpy as jnp
from jax import lax
from jax.experimental import pallas as pl
from jax.experimental.pallas import tpu as pltpu
from jax.experimental.pallas import tpu_sc as plsc

B, L, N, S = 8, 2048, 1000, 4
P = B * L
NCORES = 2
NSUB = 16
NW = NCORES * NSUB
PW = P // NW
LANES = 16
CN = 112
OFFS = [0, 112, 224, 336, 448, 560, 672, 784, 888]
WINS = [(0, 112), (112, 224), (224, 336), (336, 448), (448, 560),
        (560, 672), (672, 784), (784, 896), (896, 1000)]
IPC = PW // LANES
W_PER_USER = L // PW


def _log1p_poly(z):
    t = z / (z + 2.0)
    t2 = t * t
    p = 1.0 / 7.0 + t2 * (1.0 / 9.0)
    p = 1.0 / 5.0 + t2 * p
    p = 1.0 / 3.0 + t2 * p
    return 2.0 * t * (1.0 + t2 * p)


def _log_sigmoid(d):
    z = jnp.exp(-jnp.abs(d))
    return jnp.minimum(d, 0.0) - _log1p_poly(z)


def _body(scores, labels, negs, xlens, parts,
          lab_v, nid_sv, xl_v, chunk_v, chunk2_v, pv_v, nv_v,
          res_v, sem):
    wid = lax.axis_index("s") * NCORES + lax.axis_index("c")
    b_user = lax.div(wid, W_PER_USER)
    l0 = lax.rem(wid, W_PER_USER) * PW
    iota = lax.iota(jnp.int32, LANES)

    pltpu.sync_copy(labels.at[b_user, pl.ds(l0, PW)], lab_v)
    for s in range(S):
        pltpu.sync_copy(negs.at[b_user * S + s, pl.ds(l0, PW)], nid_sv.at[s])
    pltpu.sync_copy(xlens, xl_v)

    xlen = xl_v[wid]
    alive = jnp.logical_and(iota >= 0, l0 < xlen)
    n_alive = plsc.all_reduce_population_count(alive)

    @pl.when(n_alive[0] > 0)
    def _():
        bufs = [chunk_v, chunk2_v]
        NWIN = len(OFFS)

        def start(g):
            return pltpu.async_copy(
                scores.at[pl.ds(b_user * N + OFFS[g], CN), pl.ds(l0, PW)],
                bufs[g % 2], sem)

        start(0)
        for g in range(NWIN):
            if g + 1 < NWIN:
                start(g + 1)
            pltpu.make_async_copy(
                scores.at[pl.ds(b_user * N + OFFS[g], CN), pl.ds(l0, PW)],
                bufs[g % 2], sem).wait()
            rb = OFFS[g]
            w0, w1 = WINS[g]
            buf = bufs[g % 2]

            def harvest(i, carry, buf=buf, rb=rb, w0=w0, w1=w1):
                lcol = i * LANES + iota
                lab16 = lab_v[pl.ds(i * LANES, LANES)]
                m = jnp.logical_and(lab16 >= w0, lab16 < w1)
                nl = jnp.where(m, lab16 - rb, 0)
                v = plsc.load_gather(buf, [nl, lcol])
                pv_v[pl.ds(i * LANES, LANES)] = jnp.where(
                    m, v, pv_v[pl.ds(i * LANES, LANES)])
                for s in range(S):
                    nid16 = nid_sv[s, pl.ds(i * LANES, LANES)]
                    ms = jnp.logical_and(nid16 >= w0, nid16 < w1)
                    nls = jnp.where(ms, nid16 - rb, 0)
                    vs = plsc.load_gather(buf, [nls, lcol])
                    nv_v[s, pl.ds(i * LANES, LANES)] = jnp.where(
                        ms, vs, nv_v[s, pl.ds(i * LANES, LANES)])
                return carry

            lax.fori_loop(0, IPC, harvest, 0)

    xlenf = jnp.maximum(xlen, 1).astype(jnp.float32)
    scale = (-1.0 / B) / xlenf

    def comp(i, acc):
        pos = pv_v[pl.ds(i * LANES, LANES)]
        ls = jnp.zeros((LANES,), jnp.float32)
        for s in range(S):
            neg = nv_v[s, pl.ds(i * LANES, LANES)]
            ls = ls + _log_sigmoid(pos - neg)
        l_pos = l0 + i * LANES + iota
        return acc + jnp.where(l_pos < xlen, ls, 0.0)

    acc = lax.fori_loop(0, IPC, comp, jnp.zeros((LANES,), jnp.float32))
    res_v[...] = acc * scale
    pltpu.sync_copy(res_v, parts.at[wid])


def _body2(p_ref, o_ref):
    o_ref[...] = jnp.sum(p_ref[...]).reshape(1, 1)


@jax.jit
def _bpr_loss(output, labels, x_lens, neg_ids):
    scores = jnp.transpose(output, (0, 2, 1)).reshape(B * N, L)
    lab = labels
    neg = jnp.transpose(neg_ids, (0, 2, 1)).reshape(B * S, L)
    xl_rep = jnp.broadcast_to(
        jnp.repeat(x_lens.astype(jnp.int32), W_PER_USER)[:, None],
        (NW, LANES))
    mesh = plsc.VectorSubcoreMesh(
        core_axis_name="c", subcore_axis_name="s", num_cores=NCORES)
    params = pltpu.CompilerParams(needs_layout_passes=False)
    parts = pl.kernel(
        _body,
        out_type=jax.ShapeDtypeStruct((NW, LANES), jnp.float32),
        mesh=mesh,
        scratch_types=[
            pltpu.VMEM((PW,), jnp.int32),
            pltpu.VMEM((S, PW), jnp.int32),
            pltpu.VMEM((NW, LANES), jnp.int32),
            pltpu.VMEM((CN, PW), jnp.float32),
            pltpu.VMEM((CN, PW), jnp.float32),
            pltpu.VMEM((PW,), jnp.float32),
            pltpu.VMEM((S, PW), jnp.float32),
            pltpu.VMEM((LANES,), jnp.float32),
            pltpu.SemaphoreType.DMA,
        ],
        compiler_params=params,
    )(scores, lab, neg, xl_rep)
    res = pl.pallas_call(
        _body2,
        out_shape=jax.ShapeDtypeStruct((1, 1), jnp.float32),
    )(parts)
    return res.reshape(1)


def kernel(output, labels, x_lens, uids, neg_ids):
    return _bpr_loss(output, labels, x_lens, neg_ids)


# speedup vs baseline: 2.1028x; 1.0281x over previous
import jax
import jax.numpy as jnp
from jax import lax
from jax.experimental import pallas as pl
from jax.experimental.pallas import tpu as pltpu
from jax.experimental.pallas import tpu_sc as plsc

B, L, N, S = 8, 2048, 1000, 4
P = B * L
NCORES = 2
NSUB = 16
NW = NCORES * NSUB
PW = P // NW
LANES = 16
CN = 112
OFFS = [0, 112, 224, 336, 448, 560, 672, 784, 888]
WINS = [(0, 112), (112, 224), (224, 336), (336, 448), (448, 560),
        (560, 672), (672, 784), (784, 896), (896, 1000)]
IPC = PW // LANES
W_PER_USER = L // PW


def _log1p_poly(z):
    t = z / (z + 2.0)
    t2 = t * t
    p = 1.0 / 7.0 + t2 * (1.0 / 9.0)
    p = 1.0 / 5.0 + t2 * p
    p = 1.0 / 3.0 + t2 * p
    return 2.0 * t * (1.0 + t2 * p)


def _log_sigmoid(d):
    z = jnp.exp(-jnp.abs(d))
    return jnp.minimum(d, 0.0) - _log1p_poly(z)


def _body(scores, labels, negs, xlens, parts,
          lab_v, nid_sv, xl_v, chunk_v, chunk2_v, pv_v, nv_v,
          res_v, sem):
    wid = lax.axis_index("s") * NCORES + lax.axis_index("c")
    b_user = lax.div(wid, W_PER_USER)
    l0 = lax.rem(wid, W_PER_USER) * PW
    iota = lax.iota(jnp.int32, LANES)

    pltpu.sync_copy(labels.at[b_user, pl.ds(l0, PW)], lab_v)
    for s in range(S):
        pltpu.sync_copy(negs.at[b_user * S + s, pl.ds(l0, PW)], nid_sv.at[s])
    pltpu.sync_copy(xlens, xl_v)

    xlen = xl_v[wid]
    m1 = (l0 + iota * LANES) < xlen
    m2 = (l0 + (LANES + iota) * LANES) < xlen
    nv1 = plsc.all_reduce_population_count(m1)
    nv2 = plsc.all_reduce_population_count(m2)
    n_vregs = nv1[0] + nv2[0]

    @pl.when(n_vregs > 0)
    def _():
        bufs = [chunk_v, chunk2_v]
        NWIN = len(OFFS)

        def start(g):
            return pltpu.async_copy(
                scores.at[pl.ds(b_user * N + OFFS[g], CN), pl.ds(l0, PW)],
                bufs[g % 2], sem)

        start(0)
        for g in range(NWIN):
            if g + 1 < NWIN:
                start(g + 1)
            pltpu.make_async_copy(
                scores.at[pl.ds(b_user * N + OFFS[g], CN), pl.ds(l0, PW)],
                bufs[g % 2], sem).wait()
            rb = OFFS[g]
            w0, w1 = WINS[g]
            buf = bufs[g % 2]

            def harvest(i, carry, buf=buf, rb=rb, w0=w0, w1=w1):
                lcol = i * LANES + iota
                lab16 = lab_v[pl.ds(i * LANES, LANES)]
                m = jnp.logical_and(lab16 >= w0, lab16 < w1)
                nl = jnp.where(m, lab16 - rb, 0)
                v = plsc.load_gather(buf, [nl, lcol])
                pv_v[pl.ds(i * LANES, LANES)] = jnp.where(
                    m, v, pv_v[pl.ds(i * LANES, LANES)])
                for s in range(S):
                    nid16 = nid_sv[s, pl.ds(i * LANES, LANES)]
                    ms = jnp.logical_and(nid16 >= w0, nid16 < w1)
                    nls = jnp.where(ms, nid16 - rb, 0)
                    vs = plsc.load_gather(buf, [nls, lcol])
                    nv_v[s, pl.ds(i * LANES, LANES)] = jnp.where(
                        ms, vs, nv_v[s, pl.ds(i * LANES, LANES)])
                return carry

            lax.fori_loop(0, n_vregs, harvest, 0)

    xlenf = jnp.maximum(xlen, 1).astype(jnp.float32)
    scale = (-1.0 / B) / xlenf

    def comp(i, acc):
        pos = pv_v[pl.ds(i * LANES, LANES)]
        ls = jnp.zeros((LANES,), jnp.float32)
        for s in range(S):
            neg = nv_v[s, pl.ds(i * LANES, LANES)]
            ls = ls + _log_sigmoid(pos - neg)
        l_pos = l0 + i * LANES + iota
        return acc + jnp.where(l_pos < xlen, ls, 0.0)

    acc = lax.fori_loop(0, n_vregs, comp, jnp.zeros((LANES,), jnp.float32))
    res_v[...] = acc * scale
    pltpu.sync_copy(res_v, parts.at[wid])


def _body2(p_ref, o_ref):
    o_ref[...] = jnp.sum(p_ref[...]).reshape(1, 1)


@jax.jit
def _bpr_loss(output, labels, x_lens, neg_ids):
    scores = jnp.transpose(output, (0, 2, 1)).reshape(B * N, L)
    lab = labels
    neg = jnp.transpose(neg_ids, (0, 2, 1)).reshape(B * S, L)
    xl_rep = jnp.broadcast_to(
        jnp.repeat(x_lens.astype(jnp.int32), W_PER_USER)[:, None],
        (NW, LANES))
    mesh = plsc.VectorSubcoreMesh(
        core_axis_name="c", subcore_axis_name="s", num_cores=NCORES)
    params = pltpu.CompilerParams(needs_layout_passes=False)
    parts = pl.kernel(
        _body,
        out_type=jax.ShapeDtypeStruct((NW, LANES), jnp.float32),
        mesh=mesh,
        scratch_types=[
            pltpu.VMEM((PW,), jnp.int32),
            pltpu.VMEM((S, PW), jnp.int32),
            pltpu.VMEM((NW, LANES), jnp.int32),
            pltpu.VMEM((CN, PW), jnp.float32),
            pltpu.VMEM((CN, PW), jnp.float32),
            pltpu.VMEM((PW,), jnp.float32),
            pltpu.VMEM((S, PW), jnp.float32),
            pltpu.VMEM((LANES,), jnp.float32),
            pltpu.SemaphoreType.DMA,
        ],
        compiler_params=params,
    )(scores, lab, neg, xl_rep)
    res = pl.pallas_call(
        _body2,
        out_shape=jax.ShapeDtypeStruct((1, 1), jnp.float32),
    )(parts)
    return res.reshape(1)


def kernel(output, labels, x_lens, uids, neg_ids):
    return _bpr_loss(output, labels, x_lens, neg_ids)
